# baseline (speedup 1.0000x reference)
import functools
import math

import jax
import jax.numpy as jnp
from jax import lax
from jax.experimental import pallas as pl
from jax.experimental.pallas import tpu as pltpu
from jax.experimental.pallas import tpu_sc as plsc

N = 10000
E = 320000
D = 128
H = 4
DH = 32
R = 32
V = 100
G = 32
CUT = 6.0
GAMMA = (R / CUT) ** 2
INV_SQRT_DH = 1.0 / math.sqrt(DH)

NC, NS, NW = 2, 16, 32
EW = E // NW
C = 40
NCH = EW // C
NPS = N // NS
SLAB = 632
SLAB_LAST = N - (NS - 1) * SLAB

BN = 1000
BE = 1000
F32 = jnp.float32


def _layer_norm(t):
    m = jnp.mean(t, axis=-1, keepdims=True)
    v = jnp.mean((t - m) ** 2, axis=-1, keepdims=True)
    return (t - m) / jnp.sqrt(v + 1e-5)


def _encode_body(x_ref, posp_ref, embp_ref, winp_ref, bin_ref, h_ref):
    xb = x_ref[0, 0, :]
    onehot = (xb[:, None] == lax.broadcasted_iota(jnp.int32, (BN, 128), 1))
    he = jnp.dot(onehot.astype(F32), embp_ref[...],
                 preferred_element_type=F32)
    hp = (jnp.dot(he, winp_ref[0:32, :], preferred_element_type=F32)
          + jnp.dot(posp_ref[...], winp_ref[32:40, :],
                    preferred_element_type=F32)
          + bin_ref[0, :][None, :])
    h_ref[...] = _layer_norm(jax.nn.gelu(hp))


def _encode(x3, posp, embp, winp, bin2):
    return pl.pallas_call(
        _encode_body,
        grid=(N // BN,),
        in_specs=[
            pl.BlockSpec((1, 1, BN), lambda i: (i, 0, 0)),
            pl.BlockSpec((BN, 8), lambda i: (i, 0)),
            pl.BlockSpec((128, 32), lambda i: (0, 0)),
            pl.BlockSpec((40, 128), lambda i: (0, 0)),
            pl.BlockSpec((1, 128), lambda i: (0, 0)),
        ],
        out_specs=pl.BlockSpec((BN, 128), lambda i: (i, 0)),
        out_shape=jax.ShapeDtypeStruct((N, 128), F32),
    )(x3, posp, embp, winp, bin2)


def _qkvs_body(h_ref, wq_ref, wk_ref, wv_ref, ws_ref,
               q_ref, k_ref, v_ref, s_ref):
    hb = h_ref[...]
    q_ref[...] = jnp.dot(hb, wq_ref[...], preferred_element_type=F32)
    k_ref[...] = jnp.dot(hb, wk_ref[...], preferred_element_type=F32)
    v_ref[...] = jnp.dot(hb, wv_ref[...], preferred_element_type=F32)
    s_ref[...] = jnp.dot(hb, ws_ref[...], preferred_element_type=F32)


def _qkvs(h, wq, wk, wv, ws):
    w_spec = pl.BlockSpec((128, 128), lambda i: (0, 0))
    n_spec = pl.BlockSpec((BN, 128), lambda i: (i, 0))
    out = jax.ShapeDtypeStruct((N, 128), F32)
    return pl.pallas_call(
        _qkvs_body,
        grid=(N // BN,),
        in_specs=[n_spec, w_spec, w_spec, w_spec, w_spec],
        out_specs=[n_spec, n_spec, n_spec, n_spec],
        out_shape=[out, out, out, out],
    )(h, wq, wk, wv, ws)


def _efeat_body(d2_ref, we_ref, e_ref):
    dist = jnp.sqrt(d2_ref[0, 0, :] + 1e-12)
    cen = lax.broadcasted_iota(jnp.int32, (BE, R), 1).astype(F32) * (
        CUT / (R - 1))
    rbf = jnp.exp(-GAMMA * (dist[:, None] - cen) ** 2)
    e_ref[...] = jnp.dot(rbf, we_ref[...], preferred_element_type=F32)


def _efeat(d2_3, we):
    return pl.pallas_call(
        _efeat_body,
        grid=(E // BE,),
        in_specs=[
            pl.BlockSpec((1, 1, BE), lambda i: (i, 0, 0)),
            pl.BlockSpec((R, 128), lambda i: (0, 0)),
        ],
        out_specs=pl.BlockSpec((BE, 128), lambda i: (i, 0)),
        out_shape=jax.ShapeDtypeStruct((E, 128), F32),
    )(d2_3, we)


def _dencomb_body(denp_ref, den_ref):
    den4 = jnp.sum(denp_ref[...], axis=0) + 1e-16
    pat = (lax.broadcasted_iota(jnp.int32, (4, 128), 1) // 32
           == lax.broadcasted_iota(jnp.int32, (4, 128), 0)).astype(F32)
    den_ref[...] = jnp.dot(1.0 / den4, pat, preferred_element_type=F32)


def _dencomb(denp):
    return pl.pallas_call(
        _dencomb_body,
        grid=(N // BN,),
        in_specs=[pl.BlockSpec((NW, BN, 4), lambda i: (0, i, 0))],
        out_specs=pl.BlockSpec((BN, 128), lambda i: (i, 0)),
        out_shape=jax.ShapeDtypeStruct((N, 128), F32),
    )(denp)


def _update_body(outp_ref, invb_ref, skip_ref, wfc_ref, bfc_ref, h_ref):
    o = (outp_ref[0] + outp_ref[1]) * invb_ref[...] + skip_ref[...]
    t = jax.nn.gelu(jnp.dot(o, wfc_ref[...], preferred_element_type=F32)
                    + bfc_ref[0, :][None, :])
    h_ref[...] = _layer_norm(t)


def _update(outp, invb, skip, wfc, bfc2):
    return pl.pallas_call(
        _update_body,
        grid=(N // BN,),
        in_specs=[
            pl.BlockSpec((NC, BN, 128), lambda i: (0, i, 0)),
            pl.BlockSpec((BN, 128), lambda i: (i, 0)),
            pl.BlockSpec((BN, 128), lambda i: (i, 0)),
            pl.BlockSpec((128, 128), lambda i: (0, 0)),
            pl.BlockSpec((1, 128), lambda i: (0, 0)),
        ],
        out_specs=pl.BlockSpec((BN, 128), lambda i: (i, 0)),
        out_shape=jax.ShapeDtypeStruct((N, 128), F32),
    )(outp, invb, skip, wfc, bfc2)


def _pool_body(batch_ref, h_ref, out_ref):
    i = pl.program_id(0)
    bb = batch_ref[0, 0, :]
    onehot = (bb[:, None] == lax.broadcasted_iota(jnp.int32, (BN, G), 1))
    g = lax.dot_general(onehot.astype(F32), h_ref[...],
                        (((0,), (0,)), ((), ())),
                        preferred_element_type=F32)

    @pl.when(i == 0)
    def _():
        out_ref[...] = jnp.zeros_like(out_ref)

    out_ref[...] += g


def _pool(batch3, h):
    return pl.pallas_call(
        _pool_body,
        grid=(N // BN,),
        in_specs=[
            pl.BlockSpec((1, 1, BN), lambda i: (i, 0, 0)),
            pl.BlockSpec((BN, 128), lambda i: (i, 0)),
        ],
        out_specs=pl.BlockSpec((G, 128), lambda i: (0, 0)),
        out_shape=jax.ShapeDtypeStruct((G, 128), F32),
    )(batch3, h)


def _wid():
    return lax.axis_index("s") * NC + lax.axis_index("c")


def _io():
    return lax.broadcasted_iota(jnp.int32, (16,), 0)


_GDNUMS = lax.GatherDimensionNumbers(
    offset_dims=(), collapsed_slice_dims=(0,), start_index_map=(0,))


def _take16(v, idx16):
    return lax.gather(v, idx16[:, None], _GDNUMS, (1,),
                      mode=lax.GatherScatterMode.PROMISE_IN_BOUNDS)


def _hsum_splat(v):
    io = _io()
    for sh in (8, 4, 2, 1):
        v = v + _take16(v, jnp.bitwise_xor(io, sh))
    return v


@functools.lru_cache(maxsize=None)
def _sc_kernels():
    mesh = plsc.VectorSubcoreMesh(core_axis_name="c", subcore_axis_name="s",
                                  num_cores=NC, num_subcores=NS)
    cp = pltpu.CompilerParams(needs_layout_passes=False)
    d2_k = pl.kernel(
        _d2_body,
        out_type=jax.ShapeDtypeStruct((E,), F32),
        mesh=mesh,
        compiler_params=cp,
        scratch_types=[
            pltpu.VMEM((N,), F32), pltpu.VMEM((N,), F32),
            pltpu.VMEM((N,), F32),
            pltpu.VMEM((EW,), jnp.int32), pltpu.VMEM((EW,), jnp.int32),
            pltpu.VMEM((EW,), F32),
        ],
    )
    passa_k = pl.kernel(
        _passa_body,
        out_type=(jax.ShapeDtypeStruct((4 * E,), F32),
                  jax.ShapeDtypeStruct((NW * 4 * N,), F32)),
        mesh=mesh,
        compiler_params=cp,
        scratch_types=[
            pltpu.VMEM((EW,), jnp.int32), pltpu.VMEM((EW,), jnp.int32),
            pltpu.VMEM((C, 128), F32), pltpu.VMEM((C, 128), F32),
            pltpu.VMEM((C, 128), F32), pltpu.VMEM((C, 128), F32),
            pltpu.VMEM((C, 128), F32), pltpu.VMEM((C, 128), F32),
            pltpu.VMEM((4 * C,), F32), pltpu.VMEM((4 * C,), F32),
            pltpu.VMEM((4 * N,), F32),
            pltpu.SemaphoreType.DMA, pltpu.SemaphoreType.DMA,
            pltpu.SemaphoreType.DMA, pltpu.SemaphoreType.DMA,
        ],
    )
    passb_k = pl.kernel(
        _passb_body,
        out_type=jax.ShapeDtypeStruct((NC * N, 128), F32),
        mesh=mesh,
        compiler_params=cp,
        scratch_types=[
            pltpu.VMEM((EW,), jnp.int32),
            pltpu.VMEM((C, 128), F32), pltpu.VMEM((C, 128), F32),
            pltpu.VMEM((C, 128), F32), pltpu.VMEM((C, 128), F32),
            pltpu.VMEM((C, 128), F32), pltpu.VMEM((C, 128), F32),
            pltpu.VMEM((4 * C,), F32), pltpu.VMEM((4 * C,), F32),
            pltpu.VMEM((C,), jnp.int32), pltpu.VMEM((C,), jnp.int32),
            pltpu.VMEM_SHARED((N, 128), F32),
            pltpu.SemaphoreType.DMA, pltpu.SemaphoreType.DMA,
            pltpu.SemaphoreType.DMA, pltpu.SemaphoreType.DMA,
            pltpu.SemaphoreType.DMA, pltpu.SemaphoreType.DMA,
        ],
    )
    return d2_k, passa_k, passb_k


def _d2_body(src_e, dst_e, posx, posy, posz, d2_out,
             px, py, pz, srcv, dstv, d2v):
    wid = _wid()
    pltpu.sync_copy(posx, px)
    pltpu.sync_copy(posy, py)
    pltpu.sync_copy(posz, pz)

    ebase = wid * EW
    pltpu.sync_copy(src_e.at[pl.ds(ebase, EW)], srcv)
    pltpu.sync_copy(dst_e.at[pl.ds(ebase, EW)], dstv)

    def grp(g, _):
        i0 = g * 16
        si = srcv[pl.ds(i0, 16)]
        di = dstv[pl.ds(i0, 16)]
        dx = plsc.load_gather(px, [si]) - plsc.load_gather(px, [di])
        dy = plsc.load_gather(py, [si]) - plsc.load_gather(py, [di])
        dz = plsc.load_gather(pz, [si]) - plsc.load_gather(pz, [di])
        d2v[pl.ds(i0, 16)] = dx * dx + dy * dy + dz * dz
        return 0

    lax.fori_loop(0, EW // 16, grp, 0, unroll=4)
    pltpu.sync_copy(d2v, d2_out.at[pl.ds(ebase, EW)])


def _passa_body(src_e, dst_e, q_hbm, k_hbm, e_hbm, ex_out, denp_out,
                srcall, dstall, qv0, qv1, kv0, kv1, ev0, ev1, sv0, sv1,
                denv, si0, si1, so0, so1):
    wid = _wid()
    io = _io()
    io3 = jnp.bitwise_and(io, 3)
    m0 = io == 0
    m1 = io == 1
    m2 = io == 2
    mden = io < 4
    ebase = wid * EW
    qv, kv, ev = (qv0, qv1), (kv0, kv1), (ev0, ev1)
    sv, si, so = (sv0, sv1), (si0, si1), (so0, so1)

    pltpu.sync_copy(src_e.at[pl.ds(ebase, EW)], srcall)
    pltpu.sync_copy(dst_e.at[pl.ds(ebase, EW)], dstall)

    def zero(i, _):
        denv[pl.ds(i * 16, 16)] = jnp.zeros((16,), F32)
        return 0

    lax.fori_loop(0, (4 * N) // 16, zero, 0)

    def fire(j, b):
        off = j * C
        pltpu.async_copy(q_hbm.at[dstall.at[pl.ds(off, C)]], qv[b], si[b])
        pltpu.async_copy(k_hbm.at[srcall.at[pl.ds(off, C)]], kv[b], si[b])
        pltpu.async_copy(e_hbm.at[pl.ds(ebase + off, C), :], ev[b], si[b])

    def wait_in(j, b):
        off = j * C
        pltpu.make_async_copy(q_hbm.at[dstall.at[pl.ds(off, C)]], qv[b],
                              si[b]).wait()
        pltpu.make_async_copy(k_hbm.at[srcall.at[pl.ds(off, C)]], kv[b],
                              si[b]).wait()
        pltpu.make_async_copy(e_hbm.at[pl.ds(ebase + off, C), :], ev[b],
                              si[b]).wait()

    fire(0, 0)
    fire(1, 1)

    def pair(jj, _):
        for b in range(2):
            j = 2 * jj + b
            wait_in(j, b)

            @pl.when(jj >= 1)
            def _():
                pltpu.make_async_copy(
                    sv[b], ex_out.at[pl.ds(4 * ebase, 4 * C)], so[b]).wait()

            qb, kb, eb, svb = qv[b], kv[b], ev[b], sv[b]

            for g in (0, 16, C - 16):
                rows = io + g
                for hh in range(H):
                    a0 = a1 = None
                    for d in range(DH):
                        cols = jnp.full((16,), 32 * hh + d, jnp.int32)
                        t = (plsc.load_gather(kb, [rows, cols])
                             + plsc.load_gather(eb, [rows, cols]))
                        p = plsc.load_gather(qb, [rows, cols]) * t
                        if d % 2 == 0:
                            a0 = p if a0 is None else a0 + p
                        else:
                            a1 = p if a1 is None else a1 + p
                    exh = jnp.exp((a0 + a1) * INV_SQRT_DH)
                    plsc.store_scatter(svb, [rows * 4 + hh], exh)

            def edge(e, _):
                exrow = plsc.load_gather(
                    svb, [jnp.full((16,), 4 * e, jnp.int32) + io3])
                dstsp = plsc.load_gather(
                    dstall, [jnp.full((16,), j * C + e, jnp.int32)])
                plsc.addupdate_scatter(denv, [dstsp * 4 + io3], exrow,
                                       mask=mden)
                return 0

            lax.fori_loop(0, C, edge, 0, unroll=4)
            pltpu.async_copy(svb,
                             ex_out.at[pl.ds(4 * (ebase + j * C), 4 * C)],
                             so[b])
            fire(jnp.minimum(j + 2, NCH - 1), b)
        return 0

    lax.fori_loop(0, NCH // 2, pair, 0)
    for b in range(2):
        wait_in(0, b)
        pltpu.make_async_copy(sv[b], ex_out.at[pl.ds(4 * ebase, 4 * C)],
                              so[b]).wait()
    pltpu.sync_copy(denv, denp_out.at[pl.ds(wid * 4 * N, 4 * N)])


def _passb_body(src_e, dst_e, v_hbm, e_hbm, ex_hbm, zeros_hbm,
                outp, srcall, vv0, vv1, msgv0, msgv1, ev0, ev1, exv0, exv1,
                scidx0, scidx1, out_sh, sv0, sv1, ss0, ss1, sx0, sx1):
    cid = lax.axis_index("c")
    sid = lax.axis_index("s")
    wid = sid * NC + cid
    ebase = wid * EW
    vv, msgv = (vv0, vv1), (msgv0, msgv1)
    ev, exv = (ev0, ev1), (exv0, exv1)
    scidx, sv, ss = (scidx0, scidx1), (sv0, sv1), (ss0, ss1)
    sx = (sx0, sx1)

    pltpu.sync_copy(src_e.at[pl.ds(ebase, EW)], srcall)

    @pl.when(sid < NS - 1)
    def _():
        pltpu.sync_copy(zeros_hbm.at[pl.ds(sid * SLAB, SLAB), :],
                        out_sh.at[pl.ds(sid * SLAB, SLAB), :])

    @pl.when(sid == NS - 1)
    def _():
        pltpu.sync_copy(
            zeros_hbm.at[pl.ds((NS - 1) * SLAB, SLAB_LAST), :],
            out_sh.at[pl.ds((NS - 1) * SLAB, SLAB_LAST), :])

    plsc.subcore_barrier()

    def fire_v(j, b):
        off = j * C
        pltpu.async_copy(v_hbm.at[srcall.at[pl.ds(off, C)]], vv[b], sv[b])
        pltpu.async_copy(e_hbm.at[pl.ds(ebase + off, C), :], ev[b], sv[b])
        pltpu.async_copy(ex_hbm.at[pl.ds(4 * (ebase + off), 4 * C)],
                         exv[b], sv[b])

    def wait_v(j, b):
        off = j * C
        pltpu.make_async_copy(v_hbm.at[srcall.at[pl.ds(off, C)]], vv[b],
                              sv[b]).wait()
        pltpu.make_async_copy(e_hbm.at[pl.ds(ebase + off, C), :], ev[b],
                              sv[b]).wait()
        pltpu.make_async_copy(ex_hbm.at[pl.ds(4 * (ebase + off), 4 * C)],
                              exv[b], sv[b]).wait()

    fire_v(0, 0)
    fire_v(1, 1)

    def pair(jj, _):
        for b in range(2):
            j = 2 * jj + b
            wait_v(j, b)

            @pl.when(jj >= 1)
            def _():
                pltpu.make_async_copy(msgv[b], out_sh.at[scidx[b]],
                                      ss[b]).wait()

            pltpu.async_copy(dst_e.at[pl.ds(ebase + j * C, C)], scidx[b],
                             sx[b])
            vb, mb, eb, exb = vv[b], msgv[b], ev[b], exv[b]

            def edge(e, _):
                for hh in range(H):
                    ah = plsc.load_gather(
                        exb, [jnp.full((16,), 4 * e + hh, jnp.int32)])
                    for f in (2 * hh, 2 * hh + 1):
                        mb[e, pl.ds(16 * f, 16)] = ah * (
                            vb[e, pl.ds(16 * f, 16)]
                            + eb[e, pl.ds(16 * f, 16)])
                return 0

            lax.fori_loop(0, C, edge, 0, unroll=4)
            pltpu.make_async_copy(dst_e.at[pl.ds(ebase + j * C, C)],
                                  scidx[b], sx[b]).wait()
            pltpu.async_copy(mb, out_sh.at[scidx[b]], ss[b], add=True)
            fire_v(jnp.minimum(j + 2, NCH - 1), b)
        return 0

    lax.fori_loop(0, NCH // 2, pair, 0)
    for b in range(2):
        wait_v(0, b)
        pltpu.make_async_copy(msgv[b], out_sh.at[scidx[b]], ss[b]).wait()
    plsc.subcore_barrier()

    @pl.when(sid < NS - 1)
    def _():
        pltpu.sync_copy(out_sh.at[pl.ds(sid * SLAB, SLAB), :],
                        outp.at[pl.ds(cid * N + sid * SLAB, SLAB), :])

    @pl.when(sid == NS - 1)
    def _():
        pltpu.sync_copy(
            out_sh.at[pl.ds((NS - 1) * SLAB, SLAB_LAST), :],
            outp.at[pl.ds(cid * N + (NS - 1) * SLAB, SLAB_LAST), :])


def kernel(x, pos, edge_index, batch, emb, W_in, b_in, Wq, Wk, Wv, We,
           Wskip, Wfc, bfc):
    x3 = x.astype(jnp.int32).reshape(N // BN, 1, BN)
    batch3 = batch.astype(jnp.int32).reshape(N // BN, 1, BN)
    ei = edge_index.astype(jnp.int32)
    src_e, dst_e = ei[0], ei[1]
    posp = jnp.pad(pos, ((0, 0), (0, 5)))
    posx, posy, posz = pos[:, 0], pos[:, 1], pos[:, 2]
    embp = jnp.pad(emb, ((0, 28), (0, 0)))
    winp = jnp.pad(W_in, ((0, 5), (0, 0)))
    bin2 = b_in.reshape(1, 128)
    zeros_hbm = jnp.zeros((N, 128), F32)

    d2_k, passa_k, passb_k = _sc_kernels()
    h = _encode(x3, posp, embp, winp, bin2)
    d2 = d2_k(src_e, dst_e, posx, posy, posz)
    d2_3 = d2.reshape(E // BE, 1, BE)

    for l in range(2):
        q, k, v, skip = _qkvs(h, Wq[l], Wk[l], Wv[l], Wskip[l])
        e = _efeat(d2_3, We[l])
        ex, denp = passa_k(src_e, dst_e, q, k, e)
        invb = _dencomb(denp.reshape(NW, N, 4))
        outp = passb_k(src_e, dst_e, v, e, ex, zeros_hbm)
        h = _update(outp.reshape(NC, N, 128), invb, skip, Wfc[l],
                    bfc[l].reshape(1, 128))

    return _pool(batch3, h)


# speedup vs baseline: 2.9819x; 2.9819x over previous
import functools
import math

import jax
import jax.numpy as jnp
from jax import lax
from jax.experimental import pallas as pl
from jax.experimental.pallas import tpu as pltpu
from jax.experimental.pallas import tpu_sc as plsc

N = 10000
E = 320000
D = 128
H = 4
DH = 32
R = 32
V = 100
G = 32
CUT = 6.0
GAMMA = (R / CUT) ** 2
INV_SQRT_DH = 1.0 / math.sqrt(DH)

NC, NS, NW = 2, 16, 32
EW = E // NW
C = 40
NCH = EW // C
NPS = N // NS
SLAB = 632
SLAB_LAST = N - (NS - 1) * SLAB

BN = 1000
BE = 1000
F32 = jnp.float32


def _layer_norm(t):
    m = jnp.mean(t, axis=-1, keepdims=True)
    v = jnp.mean((t - m) ** 2, axis=-1, keepdims=True)
    return (t - m) / jnp.sqrt(v + 1e-5)


def _encode_body(x_ref, posp_ref, embp_ref, winp_ref, bin_ref, h_ref):
    xb = x_ref[0, 0, :]
    onehot = (xb[:, None] == lax.broadcasted_iota(jnp.int32, (BN, 128), 1))
    he = jnp.dot(onehot.astype(F32), embp_ref[...],
                 preferred_element_type=F32)
    hp = (jnp.dot(he, winp_ref[0:32, :], preferred_element_type=F32)
          + jnp.dot(posp_ref[...], winp_ref[32:40, :],
                    preferred_element_type=F32)
          + bin_ref[0, :][None, :])
    h_ref[...] = _layer_norm(jax.nn.gelu(hp))


def _encode(x3, posp, embp, winp, bin2):
    return pl.pallas_call(
        _encode_body,
        grid=(N // BN,),
        in_specs=[
            pl.BlockSpec((1, 1, BN), lambda i: (i, 0, 0)),
            pl.BlockSpec((BN, 8), lambda i: (i, 0)),
            pl.BlockSpec((128, 32), lambda i: (0, 0)),
            pl.BlockSpec((40, 128), lambda i: (0, 0)),
            pl.BlockSpec((1, 128), lambda i: (0, 0)),
        ],
        out_specs=pl.BlockSpec((BN, 128), lambda i: (i, 0)),
        out_shape=jax.ShapeDtypeStruct((N, 128), F32),
    )(x3, posp, embp, winp, bin2)


def _qkvs_body(h_ref, wq_ref, wk_ref, wv_ref, ws_ref,
               q_ref, k_ref, v_ref, s_ref):
    hb = h_ref[...]
    q_ref[...] = jnp.dot(hb, wq_ref[...], preferred_element_type=F32)
    k_ref[...] = jnp.dot(hb, wk_ref[...], preferred_element_type=F32)
    v_ref[...] = jnp.dot(hb, wv_ref[...], preferred_element_type=F32)
    s_ref[...] = jnp.dot(hb, ws_ref[...], preferred_element_type=F32)


def _qkvs(h, wq, wk, wv, ws):
    w_spec = pl.BlockSpec((128, 128), lambda i: (0, 0))
    n_spec = pl.BlockSpec((BN, 128), lambda i: (i, 0))
    out = jax.ShapeDtypeStruct((N, 128), F32)
    return pl.pallas_call(
        _qkvs_body,
        grid=(N // BN,),
        in_specs=[n_spec, w_spec, w_spec, w_spec, w_spec],
        out_specs=[n_spec, n_spec, n_spec, n_spec],
        out_shape=[out, out, out, out],
    )(h, wq, wk, wv, ws)


def _efeat_body(d2_ref, we_ref, e_ref):
    dist = jnp.sqrt(d2_ref[0, 0, :] + 1e-12)
    cen = lax.broadcasted_iota(jnp.int32, (BE, R), 1).astype(F32) * (
        CUT / (R - 1))
    rbf = jnp.exp(-GAMMA * (dist[:, None] - cen) ** 2)
    e_ref[...] = jnp.dot(rbf, we_ref[...], preferred_element_type=F32)


def _efeat(d2_3, we):
    return pl.pallas_call(
        _efeat_body,
        grid=(E // BE,),
        in_specs=[
            pl.BlockSpec((1, 1, BE), lambda i: (i, 0, 0)),
            pl.BlockSpec((R, 128), lambda i: (0, 0)),
        ],
        out_specs=pl.BlockSpec((BE, 128), lambda i: (i, 0)),
        out_shape=jax.ShapeDtypeStruct((E, 128), F32),
    )(d2_3, we)


def _dencomb_body(denp_ref, den_ref):
    den4 = jnp.sum(denp_ref[...], axis=0) + 1e-16
    pat = (lax.broadcasted_iota(jnp.int32, (4, 128), 1) // 32
           == lax.broadcasted_iota(jnp.int32, (4, 128), 0)).astype(F32)
    den_ref[...] = jnp.dot(1.0 / den4, pat, preferred_element_type=F32)


def _dencomb(denp):
    return pl.pallas_call(
        _dencomb_body,
        grid=(N // BN,),
        in_specs=[pl.BlockSpec((NW, BN, 4), lambda i: (0, i, 0))],
        out_specs=pl.BlockSpec((BN, 128), lambda i: (i, 0)),
        out_shape=jax.ShapeDtypeStruct((N, 128), F32),
    )(denp)


def _update_body(outp_ref, invb_ref, skip_ref, wfc_ref, bfc_ref, h_ref):
    o = (outp_ref[0] + outp_ref[1]) * invb_ref[...] + skip_ref[...]
    t = jax.nn.gelu(jnp.dot(o, wfc_ref[...], preferred_element_type=F32)
                    + bfc_ref[0, :][None, :])
    h_ref[...] = _layer_norm(t)


def _update(outp, invb, skip, wfc, bfc2):
    return pl.pallas_call(
        _update_body,
        grid=(N // BN,),
        in_specs=[
            pl.BlockSpec((NC, BN, 128), lambda i: (0, i, 0)),
            pl.BlockSpec((BN, 128), lambda i: (i, 0)),
            pl.BlockSpec((BN, 128), lambda i: (i, 0)),
            pl.BlockSpec((128, 128), lambda i: (0, 0)),
            pl.BlockSpec((1, 128), lambda i: (0, 0)),
        ],
        out_specs=pl.BlockSpec((BN, 128), lambda i: (i, 0)),
        out_shape=jax.ShapeDtypeStruct((N, 128), F32),
    )(outp, invb, skip, wfc, bfc2)


def _pool_body(batch_ref, h_ref, out_ref):
    i = pl.program_id(0)
    bb = batch_ref[0, 0, :]
    onehot = (bb[:, None] == lax.broadcasted_iota(jnp.int32, (BN, G), 1))
    g = lax.dot_general(onehot.astype(F32), h_ref[...],
                        (((0,), (0,)), ((), ())),
                        preferred_element_type=F32)

    @pl.when(i == 0)
    def _():
        out_ref[...] = jnp.zeros_like(out_ref)

    out_ref[...] += g


def _pool(batch3, h):
    return pl.pallas_call(
        _pool_body,
        grid=(N // BN,),
        in_specs=[
            pl.BlockSpec((1, 1, BN), lambda i: (i, 0, 0)),
            pl.BlockSpec((BN, 128), lambda i: (i, 0)),
        ],
        out_specs=pl.BlockSpec((G, 128), lambda i: (0, 0)),
        out_shape=jax.ShapeDtypeStruct((G, 128), F32),
    )(batch3, h)


def _wid():
    return lax.axis_index("s") * NC + lax.axis_index("c")


def _io():
    return lax.broadcasted_iota(jnp.int32, (16,), 0)


_GDNUMS = lax.GatherDimensionNumbers(
    offset_dims=(), collapsed_slice_dims=(0,), start_index_map=(0,))


def _take16(v, idx16):
    return lax.gather(v, idx16[:, None], _GDNUMS, (1,),
                      mode=lax.GatherScatterMode.PROMISE_IN_BOUNDS)


def _hsum_splat(v):
    io = _io()
    for sh in (8, 4, 2, 1):
        v = v + _take16(v, jnp.bitwise_xor(io, sh))
    return v


@functools.lru_cache(maxsize=None)
def _sc_kernels():
    mesh = plsc.VectorSubcoreMesh(core_axis_name="c", subcore_axis_name="s",
                                  num_cores=NC, num_subcores=NS)
    cp = pltpu.CompilerParams(needs_layout_passes=False)
    d2_k = pl.kernel(
        _d2_body,
        out_type=jax.ShapeDtypeStruct((E,), F32),
        mesh=mesh,
        compiler_params=cp,
        scratch_types=[
            pltpu.VMEM((N,), F32), pltpu.VMEM((N,), F32),
            pltpu.VMEM((N,), F32),
            pltpu.VMEM((EW,), jnp.int32), pltpu.VMEM((EW,), jnp.int32),
            pltpu.VMEM((EW,), F32),
        ],
    )
    passa_k = pl.kernel(
        _passa_body,
        out_type=(jax.ShapeDtypeStruct((4 * E,), F32),
                  jax.ShapeDtypeStruct((NW * 4 * N,), F32)),
        mesh=mesh,
        compiler_params=cp,
        scratch_types=[
            pltpu.VMEM((EW,), jnp.int32), pltpu.VMEM((EW,), jnp.int32),
            pltpu.VMEM((C, 128), F32), pltpu.VMEM((C, 128), F32),
            pltpu.VMEM((C, 128), F32), pltpu.VMEM((C, 128), F32),
            pltpu.VMEM((C, 128), F32), pltpu.VMEM((C, 128), F32),
            pltpu.VMEM((4 * C,), F32), pltpu.VMEM((4 * C,), F32),
            pltpu.VMEM((4 * N,), F32),
            pltpu.SemaphoreType.DMA, pltpu.SemaphoreType.DMA,
            pltpu.SemaphoreType.DMA, pltpu.SemaphoreType.DMA,
        ],
    )
    passb_k = pl.kernel(
        _passb_body,
        out_type=jax.ShapeDtypeStruct((NC * N, 128), F32),
        mesh=mesh,
        compiler_params=cp,
        scratch_types=[
            pltpu.VMEM((EW,), jnp.int32),
            pltpu.VMEM((C, 128), F32), pltpu.VMEM((C, 128), F32),
            pltpu.VMEM((C, 128), F32), pltpu.VMEM((C, 128), F32),
            pltpu.VMEM((C, 128), F32), pltpu.VMEM((C, 128), F32),
            pltpu.VMEM((4 * C,), F32), pltpu.VMEM((4 * C,), F32),
            pltpu.VMEM((C,), jnp.int32), pltpu.VMEM((C,), jnp.int32),
            pltpu.VMEM_SHARED((N, 128), F32),
            pltpu.SemaphoreType.DMA, pltpu.SemaphoreType.DMA,
            pltpu.SemaphoreType.DMA, pltpu.SemaphoreType.DMA,
            pltpu.SemaphoreType.DMA, pltpu.SemaphoreType.DMA,
        ],
    )
    return d2_k, passa_k, passb_k


def _d2_body(src_e, dst_e, posx, posy, posz, d2_out,
             px, py, pz, srcv, dstv, d2v):
    wid = _wid()
    pltpu.sync_copy(posx, px)
    pltpu.sync_copy(posy, py)
    pltpu.sync_copy(posz, pz)

    ebase = wid * EW
    pltpu.sync_copy(src_e.at[pl.ds(ebase, EW)], srcv)
    pltpu.sync_copy(dst_e.at[pl.ds(ebase, EW)], dstv)

    def grp(g, _):
        i0 = g * 16
        si = srcv[pl.ds(i0, 16)]
        di = dstv[pl.ds(i0, 16)]
        dx = plsc.load_gather(px, [si]) - plsc.load_gather(px, [di])
        dy = plsc.load_gather(py, [si]) - plsc.load_gather(py, [di])
        dz = plsc.load_gather(pz, [si]) - plsc.load_gather(pz, [di])
        d2v[pl.ds(i0, 16)] = dx * dx + dy * dy + dz * dz
        return 0

    lax.fori_loop(0, EW // 16, grp, 0, unroll=4)
    pltpu.sync_copy(d2v, d2_out.at[pl.ds(ebase, EW)])


def _passa_body(src_e, dst_e, q_hbm, k_hbm, e_hbm, ex_out, denp_out,
                srcall, dstall, qv0, qv1, kv0, kv1, ev0, ev1, sv0, sv1,
                denv, si0, si1, so0, so1):
    wid = _wid()
    io = _io()
    io3 = jnp.bitwise_and(io, 3)
    m0 = io == 0
    m1 = io == 1
    m2 = io == 2
    mden = io < 4
    ebase = wid * EW
    qv, kv, ev = (qv0, qv1), (kv0, kv1), (ev0, ev1)
    sv, si, so = (sv0, sv1), (si0, si1), (so0, so1)

    pltpu.sync_copy(src_e.at[pl.ds(ebase, EW)], srcall)
    pltpu.sync_copy(dst_e.at[pl.ds(ebase, EW)], dstall)

    def zero(i, _):
        denv[pl.ds(i * 16, 16)] = jnp.zeros((16,), F32)
        return 0

    lax.fori_loop(0, (4 * N) // 16, zero, 0)

    def fire(j, b):
        off = j * C
        pltpu.async_copy(q_hbm.at[dstall.at[pl.ds(off, C)]], qv[b], si[b])
        pltpu.async_copy(k_hbm.at[srcall.at[pl.ds(off, C)]], kv[b], si[b])
        pltpu.async_copy(e_hbm.at[pl.ds(ebase + off, C), :], ev[b], si[b])

    def wait_in(j, b):
        off = j * C
        pltpu.make_async_copy(q_hbm.at[dstall.at[pl.ds(off, C)]], qv[b],
                              si[b]).wait()
        pltpu.make_async_copy(k_hbm.at[srcall.at[pl.ds(off, C)]], kv[b],
                              si[b]).wait()
        pltpu.make_async_copy(e_hbm.at[pl.ds(ebase + off, C), :], ev[b],
                              si[b]).wait()

    fire(0, 0)
    fire(1, 1)

    def pair(jj, _):
        for b in range(2):
            j = 2 * jj + b
            wait_in(j, b)

            @pl.when(jj >= 1)
            def _():
                pltpu.make_async_copy(
                    sv[b], ex_out.at[pl.ds(4 * ebase, 4 * C)], so[b]).wait()

            qb, kb, eb, svb = qv[b], kv[b], ev[b], sv[b]

            def edge(e, _):
                accs = []
                for hh in range(H):
                    a = None
                    for f in (2 * hh, 2 * hh + 1):
                        t = kb[e, pl.ds(16 * f, 16)] + eb[e, pl.ds(16 * f, 16)]
                        p = qb[e, pl.ds(16 * f, 16)] * t
                        a = p if a is None else a + p
                    accs.append(jnp.full((16,), jnp.sum(a), F32))
                row = jnp.where(m0, accs[0],
                                jnp.where(m1, accs[1],
                                          jnp.where(m2, accs[2], accs[3])))
                exrow = jnp.exp(row * INV_SQRT_DH)
                plsc.store_scatter(svb, [jnp.full((16,), 4 * e, jnp.int32)
                                         + io3], exrow, mask=mden)
                dstsp = plsc.load_gather(
                    dstall, [jnp.full((16,), j * C + e, jnp.int32)])
                plsc.addupdate_scatter(denv, [dstsp * 4 + io3], exrow,
                                       mask=mden)
                return 0

            lax.fori_loop(0, C, edge, 0, unroll=4)
            pltpu.async_copy(svb,
                             ex_out.at[pl.ds(4 * (ebase + j * C), 4 * C)],
                             so[b])
            fire(jnp.minimum(j + 2, NCH - 1), b)
        return 0

    lax.fori_loop(0, NCH // 2, pair, 0)
    for b in range(2):
        wait_in(0, b)
        pltpu.make_async_copy(sv[b], ex_out.at[pl.ds(4 * ebase, 4 * C)],
                              so[b]).wait()
    pltpu.sync_copy(denv, denp_out.at[pl.ds(wid * 4 * N, 4 * N)])


def _passb_body(src_e, dst_e, v_hbm, e_hbm, ex_hbm, zeros_hbm,
                outp, srcall, vv0, vv1, msgv0, msgv1, ev0, ev1, exv0, exv1,
                scidx0, scidx1, out_sh, sv0, sv1, ss0, ss1, sx0, sx1):
    cid = lax.axis_index("c")
    sid = lax.axis_index("s")
    wid = sid * NC + cid
    ebase = wid * EW
    vv, msgv = (vv0, vv1), (msgv0, msgv1)
    ev, exv = (ev0, ev1), (exv0, exv1)
    scidx, sv, ss = (scidx0, scidx1), (sv0, sv1), (ss0, ss1)
    sx = (sx0, sx1)

    pltpu.sync_copy(src_e.at[pl.ds(ebase, EW)], srcall)

    @pl.when(sid < NS - 1)
    def _():
        pltpu.sync_copy(zeros_hbm.at[pl.ds(sid * SLAB, SLAB), :],
                        out_sh.at[pl.ds(sid * SLAB, SLAB), :])

    @pl.when(sid == NS - 1)
    def _():
        pltpu.sync_copy(
            zeros_hbm.at[pl.ds((NS - 1) * SLAB, SLAB_LAST), :],
            out_sh.at[pl.ds((NS - 1) * SLAB, SLAB_LAST), :])

    plsc.subcore_barrier()

    def fire_v(j, b):
        off = j * C
        pltpu.async_copy(v_hbm.at[srcall.at[pl.ds(off, C)]], vv[b], sv[b])
        pltpu.async_copy(e_hbm.at[pl.ds(ebase + off, C), :], ev[b], sv[b])
        pltpu.async_copy(ex_hbm.at[pl.ds(4 * (ebase + off), 4 * C)],
                         exv[b], sv[b])

    def wait_v(j, b):
        off = j * C
        pltpu.make_async_copy(v_hbm.at[srcall.at[pl.ds(off, C)]], vv[b],
                              sv[b]).wait()
        pltpu.make_async_copy(e_hbm.at[pl.ds(ebase + off, C), :], ev[b],
                              sv[b]).wait()
        pltpu.make_async_copy(ex_hbm.at[pl.ds(4 * (ebase + off), 4 * C)],
                              exv[b], sv[b]).wait()

    fire_v(0, 0)
    fire_v(1, 1)

    def pair(jj, _):
        for b in range(2):
            j = 2 * jj + b
            wait_v(j, b)

            @pl.when(jj >= 1)
            def _():
                pltpu.make_async_copy(msgv[b], out_sh.at[scidx[b]],
                                      ss[b]).wait()

            pltpu.async_copy(dst_e.at[pl.ds(ebase + j * C, C)], scidx[b],
                             sx[b])
            vb, mb, eb, exb = vv[b], msgv[b], ev[b], exv[b]

            def edge(e, _):
                for hh in range(H):
                    ah = plsc.load_gather(
                        exb, [jnp.full((16,), 4 * e + hh, jnp.int32)])
                    for f in (2 * hh, 2 * hh + 1):
                        mb[e, pl.ds(16 * f, 16)] = ah * (
                            vb[e, pl.ds(16 * f, 16)]
                            + eb[e, pl.ds(16 * f, 16)])
                return 0

            lax.fori_loop(0, C, edge, 0, unroll=4)
            pltpu.make_async_copy(dst_e.at[pl.ds(ebase + j * C, C)],
                                  scidx[b], sx[b]).wait()
            pltpu.async_copy(mb, out_sh.at[scidx[b]], ss[b], add=True)
            fire_v(jnp.minimum(j + 2, NCH - 1), b)
        return 0

    lax.fori_loop(0, NCH // 2, pair, 0)
    for b in range(2):
        wait_v(0, b)
        pltpu.make_async_copy(msgv[b], out_sh.at[scidx[b]], ss[b]).wait()
    plsc.subcore_barrier()

    @pl.when(sid < NS - 1)
    def _():
        pltpu.sync_copy(out_sh.at[pl.ds(sid * SLAB, SLAB), :],
                        outp.at[pl.ds(cid * N + sid * SLAB, SLAB), :])

    @pl.when(sid == NS - 1)
    def _():
        pltpu.sync_copy(
            out_sh.at[pl.ds((NS - 1) * SLAB, SLAB_LAST), :],
            outp.at[pl.ds(cid * N + (NS - 1) * SLAB, SLAB_LAST), :])


def kernel(x, pos, edge_index, batch, emb, W_in, b_in, Wq, Wk, Wv, We,
           Wskip, Wfc, bfc):
    x3 = x.astype(jnp.int32).reshape(N // BN, 1, BN)
    batch3 = batch.astype(jnp.int32).reshape(N // BN, 1, BN)
    ei = edge_index.astype(jnp.int32)
    src_e, dst_e = ei[0], ei[1]
    posp = jnp.pad(pos, ((0, 0), (0, 5)))
    posx, posy, posz = pos[:, 0], pos[:, 1], pos[:, 2]
    embp = jnp.pad(emb, ((0, 28), (0, 0)))
    winp = jnp.pad(W_in, ((0, 5), (0, 0)))
    bin2 = b_in.reshape(1, 128)
    zeros_hbm = jnp.zeros((N, 128), F32)

    d2_k, passa_k, passb_k = _sc_kernels()
    h = _encode(x3, posp, embp, winp, bin2)
    d2 = d2_k(src_e, dst_e, posx, posy, posz)
    d2_3 = d2.reshape(E // BE, 1, BE)

    for l in range(2):
        q, k, v, skip = _qkvs(h, Wq[l], Wk[l], Wv[l], Wskip[l])
        e = _efeat(d2_3, We[l])
        ex, denp = passa_k(src_e, dst_e, q, k, e)
        invb = _dencomb(denp.reshape(NW, N, 4))
        outp = passb_k(src_e, dst_e, v, e, ex, zeros_hbm)
        h = _update(outp.reshape(NC, N, 128), invb, skip, Wfc[l],
                    bfc[l].reshape(1, 128))

    return _pool(batch3, h)
